# baseline (speedup 1.0000x reference)
import functools

import jax
import jax.numpy as jnp
from jax import lax
from jax.experimental import pallas as pl
from jax.experimental.pallas import tpu as pltpu
from jax.experimental.pallas import tpu_sc as plsc

N = 10000
E = 320000
F_IN = 128
HID = 256
NCLS = 64

NC = 2
NS = 16
C = 100
KB = 5
EROWS = E // (KB * C)
OUTER = EROWS // NS
NP = 10240
RPT = NP // NS

BLK = 2000

_MESH = plsc.VectorSubcoreMesh(core_axis_name="c", subcore_axis_name="s")


def _make_sc_agg(width, with_cnt, edge_split=False):
    out_type = [jax.ShapeDtypeStruct((NC, NP, width), jnp.float32)]
    scratch = [
        pltpu.VMEM((2, KB, C), jnp.int32),
        pltpu.VMEM((2, KB, C), jnp.int32),
        pltpu.VMEM((2, KB, C, width), jnp.float32),
        pltpu.VMEM_SHARED((NP, width), jnp.float32),
        pltpu.SemaphoreType.DMA,
        pltpu.SemaphoreType.DMA,
        pltpu.SemaphoreType.DMA,
        pltpu.SemaphoreType.DMA,
    ]
    if with_cnt:
        out_type.append(jax.ShapeDtypeStruct((NC, NP, 8), jnp.float32))
        scratch.append(pltpu.VMEM((KB, C, 8), jnp.float32))
        scratch.append(pltpu.VMEM_SHARED((NP, 8), jnp.float32))
        scratch.append(pltpu.SemaphoreType.DMA)

    def body(*refs):
        if with_cnt:
            (t0_hbm, t1_hbm, src_hbm, dst_hbm, z_hbm, z8_hbm,
             ones_hbm, acc_out, cnt_out, sidx, didx, rows, acc_sh,
             gsem0, gsem1, asem0, asem1, ones_v, cnt_sh, csem) = refs
        elif edge_split:
            (t0_hbm, src_hbm, dst_hbm, z_hbm,
             acc_out, sidx, didx, rows, acc_sh,
             gsem0, gsem1, asem0, asem1) = refs
        else:
            (t0_hbm, t1_hbm, src_hbm, dst_hbm, z_hbm,
             acc_out, sidx, didx, rows, acc_sh,
             gsem0, gsem1, asem0, asem1) = refs
        gsem = (gsem0, gsem1)
        asem = (asem0, asem1)
        ci = lax.axis_index("c")
        si = lax.axis_index("s")
        r0 = si * RPT
        if edge_split:
            n_steps = EROWS // (NC * NS)
            base = (ci * NS + si) * n_steps
        else:
            n_steps = OUTER
            base = si * OUTER
        def zbody(k, carry):
            pltpu.sync_copy(z_hbm, acc_sh.at[pl.ds(r0 + k * 64, 64)])
            if with_cnt:
                pltpu.sync_copy(z8_hbm, cnt_sh.at[pl.ds(r0 + k * 64, 64)])
            return carry
        lax.fori_loop(0, RPT // 64, zbody, 0)
        if with_cnt:
            pltpu.sync_copy(ones_hbm, ones_v)
        plsc.subcore_barrier()

        def drain(sem, dst_ref, src_ref):
            pltpu.make_async_copy(src_ref, dst_ref, sem).wait()

        def run(tab_hbm):
            dummy = tab_hbm.at[pl.ds(0, C)]

            def fire_gathers(g, s):
                for j in range(KB):
                    pltpu.async_copy(tab_hbm.at[sidx.at[s, j]],
                                     rows.at[s, j], gsem[s])

            def fire_adds(s):
                for j in range(KB):
                    pltpu.async_copy(rows.at[s, j], acc_sh.at[didx.at[s, j]],
                                     asem[s], add=True)
                if with_cnt:
                    for j in range(KB):
                        pltpu.async_copy(ones_v.at[j],
                                         cnt_sh.at[didx.at[s, j]], csem,
                                         add=True)

            def sub(g, cur, drain_guard, next_guard):
                nxt = 1 - cur

                def start_next():
                    pltpu.sync_copy(src_hbm.at[base + g + 1], sidx.at[nxt])
                    pltpu.sync_copy(dst_hbm.at[base + g + 1], didx.at[nxt])
                    fire_gathers(g + 1, nxt)

                def drain_prev():
                    for j in range(KB):
                        drain(asem[nxt], rows.at[nxt, j], dummy)

                if drain_guard is None:
                    drain_prev()
                else:
                    pl.when(drain_guard)(drain_prev)
                if next_guard is None:
                    start_next()
                else:
                    pl.when(next_guard)(start_next)
                for j in range(KB):
                    drain(gsem[cur], rows.at[cur, j], dummy)
                fire_adds(cur)

            pltpu.sync_copy(src_hbm.at[base], sidx.at[0])
            pltpu.sync_copy(dst_hbm.at[base], didx.at[0])
            fire_gathers(0, 0)

            def fbody(b, carry):
                sub(2 * b, 0, drain_guard=b > 0, next_guard=None)
                sub(2 * b + 1, 1, drain_guard=None,
                    next_guard=b < n_steps // 2 - 1)
                return carry

            lax.fori_loop(0, n_steps // 2, fbody, 0)
            for j in range(KB):
                drain(asem[1], rows.at[1, j], dummy)

        if edge_split:
            run(t0_hbm)
        else:
            @pl.when(ci == 0)
            def _():
                run(t0_hbm)

            @pl.when(ci == 1)
            def _():
                run(t1_hbm)

        if with_cnt:
            def cdrain(o, carry):
                drain(csem, ones_v, ones_hbm)
                return carry
            lax.fori_loop(0, n_steps, cdrain, 0)

        plsc.subcore_barrier()
        pltpu.sync_copy(acc_sh.at[pl.ds(r0, RPT)],
                        acc_out.at[ci, pl.ds(r0, RPT)])
        if with_cnt:
            pltpu.sync_copy(cnt_sh.at[pl.ds(r0, RPT)],
                            cnt_out.at[ci, pl.ds(r0, RPT)])

    return functools.partial(
        pl.kernel, out_type=out_type, mesh=_MESH, scratch_types=scratch,
        compiler_params=pltpu.CompilerParams(use_tc_tiling_on_sc=False))(body)


_sc_agg1 = _make_sc_agg(F_IN // 2, True)
_sc_agg2 = _make_sc_agg(NCLS, False, edge_split=True)


def _dense1_body(a0_ref, a1_ref, c0_ref, c1_ref, x_ref, w1l_ref, w1r_ref,
                 b1_ref, w2_ref, b2_ref, p_ref, q_ref, inv_ref):
    cnt = (c0_ref[0, :, 0:1] + c1_ref[0, :, 0:1]) * 0.5
    inv = 1.0 / jnp.maximum(cnt, 1.0)
    agg = jnp.concatenate([a0_ref[0], a1_ref[0]], axis=1)
    mean = agg * inv
    h = jnp.maximum(
        jnp.dot(mean, w1l_ref[...], preferred_element_type=jnp.float32)
        + jnp.dot(x_ref[...], w1r_ref[...], preferred_element_type=jnp.float32)
        + b1_ref[...], 0.0)
    pq = jnp.dot(h, w2_ref[...], preferred_element_type=jnp.float32)
    p_ref[...] = pq[:, :NCLS]
    q_ref[...] = pq[:, NCLS:] + b2_ref[...]
    inv_ref[...] = jnp.broadcast_to(inv, (BLK, 8))


def _dense1(acc_parts, cnt_parts, x, W1_l, W1_r, b1, W2cat, b2):
    grid = (N // BLK,)
    return pl.pallas_call(
        _dense1_body,
        grid=grid,
        in_specs=[
            pl.BlockSpec((1, BLK, F_IN // 2), lambda i: (0, i, 0)),
            pl.BlockSpec((1, BLK, F_IN // 2), lambda i: (1, i, 0)),
            pl.BlockSpec((1, BLK, 8), lambda i: (0, i, 0)),
            pl.BlockSpec((1, BLK, 8), lambda i: (1, i, 0)),
            pl.BlockSpec((BLK, F_IN), lambda i: (i, 0)),
            pl.BlockSpec((F_IN, HID), lambda i: (0, 0)),
            pl.BlockSpec((F_IN, HID), lambda i: (0, 0)),
            pl.BlockSpec((1, HID), lambda i: (0, 0)),
            pl.BlockSpec((HID, 2 * NCLS), lambda i: (0, 0)),
            pl.BlockSpec((1, NCLS), lambda i: (0, 0)),
        ],
        out_specs=[
            pl.BlockSpec((BLK, NCLS), lambda i: (i, 0)),
            pl.BlockSpec((BLK, NCLS), lambda i: (i, 0)),
            pl.BlockSpec((BLK, 8), lambda i: (i, 0)),
        ],
        out_shape=[
            jax.ShapeDtypeStruct((N, NCLS), jnp.float32),
            jax.ShapeDtypeStruct((N, NCLS), jnp.float32),
            jax.ShapeDtypeStruct((N, 8), jnp.float32),
        ],
    )(acc_parts, acc_parts, cnt_parts, cnt_parts, x, W1_l, W1_r, b1, W2cat, b2)


def _dense2_body(a0_ref, a1_ref, inv_ref, q_ref, out_ref):
    agg = a0_ref[0] + a1_ref[0]
    z = agg * inv_ref[:, 0:1] + q_ref[...]
    m = jnp.max(z, axis=1, keepdims=True)
    s = jnp.sum(jnp.exp(z - m), axis=1, keepdims=True)
    out_ref[...] = z - m - jnp.log(s)


def _dense2(agg2_parts, inv, q):
    grid = (N // BLK,)
    return pl.pallas_call(
        _dense2_body,
        grid=grid,
        in_specs=[
            pl.BlockSpec((1, BLK, NCLS), lambda i: (0, i, 0)),
            pl.BlockSpec((1, BLK, NCLS), lambda i: (1, i, 0)),
            pl.BlockSpec((BLK, 8), lambda i: (i, 0)),
            pl.BlockSpec((BLK, NCLS), lambda i: (i, 0)),
        ],
        out_specs=pl.BlockSpec((BLK, NCLS), lambda i: (i, 0)),
        out_shape=jax.ShapeDtypeStruct((N, NCLS), jnp.float32),
    )(agg2_parts, agg2_parts, inv, q)


def kernel(x, edge_index, W1_l, W1_r, b1, W2_l, W2_r, b2):
    src = edge_index[0].reshape(EROWS, KB, C)
    dst = edge_index[1].reshape(EROWS, KB, C)

    x0 = x[:, :F_IN // 2]
    x1 = x[:, F_IN // 2:]

    z64 = jnp.zeros((64, F_IN // 2), jnp.float32)
    z8 = jnp.zeros((64, 8), jnp.float32)
    zp = jnp.zeros((64, NCLS), jnp.float32)
    ones = jnp.ones((KB, C, 8), jnp.float32)

    acc_parts, cnt_parts = jax.tree.leaves(
        _sc_agg1(x0, x1, src, dst, z64, z8, ones))

    W2cat = jnp.concatenate([W2_l, W2_r], axis=1)
    p, q, inv = _dense1(acc_parts, cnt_parts, x, W1_l, W1_r, b1[None, :],
                        W2cat, b2[None, :])

    agg2_parts, = jax.tree.leaves(_sc_agg2(p, src, dst, zp))

    return _dense2(agg2_parts, inv, q)


# speedup vs baseline: 1.0911x; 1.0911x over previous
import functools

import jax
import jax.numpy as jnp
from jax import lax
from jax.experimental import pallas as pl
from jax.experimental.pallas import tpu as pltpu
from jax.experimental.pallas import tpu_sc as plsc

N = 10000
E = 320000
F_IN = 128
HID = 256
NCLS = 64

NC = 2
NS = 16
C = 100
KB = 5
EROWS = E // (KB * C)
OUTER = EROWS // NS
NP = 10240
RPT = NP // NS

BLK = 2000

_MESH = plsc.VectorSubcoreMesh(core_axis_name="c", subcore_axis_name="s")


def _make_sc_agg(width, with_cnt, edge_split=False):
    out_type = [jax.ShapeDtypeStruct((NC, NP, width), jnp.float32)]
    scratch = [
        pltpu.VMEM((2, KB, C), jnp.int32),
        pltpu.VMEM((2, KB, C), jnp.int32),
        pltpu.VMEM((2, KB, C, width), jnp.float32),
        pltpu.VMEM_SHARED((NP, width), jnp.float32),
        pltpu.SemaphoreType.DMA,
        pltpu.SemaphoreType.DMA,
        pltpu.SemaphoreType.DMA,
        pltpu.SemaphoreType.DMA,
    ]
    if with_cnt:
        out_type.append(jax.ShapeDtypeStruct((NC, NP, 8), jnp.float32))
        scratch.append(pltpu.VMEM((KB, C, 8), jnp.float32))
        scratch.append(pltpu.VMEM_SHARED((NP, 8), jnp.float32))
        scratch.append(pltpu.SemaphoreType.DMA)

    def body(*refs):
        if with_cnt:
            (t0_hbm, t1_hbm, src_hbm, dst_hbm, za_hbm, dz_hbm, zc_hbm,
             ones_hbm, acc_out, cnt_out, sidx, didx, rows, acc_sh,
             gsem0, gsem1, asem0, asem1, ones_v, cnt_sh, csem) = refs
        elif edge_split:
            (t0_hbm, src_hbm, dst_hbm, za_hbm, dz_hbm,
             acc_out, sidx, didx, rows, acc_sh,
             gsem0, gsem1, asem0, asem1) = refs
        else:
            (t0_hbm, t1_hbm, src_hbm, dst_hbm, za_hbm, dz_hbm,
             acc_out, sidx, didx, rows, acc_sh,
             gsem0, gsem1, asem0, asem1) = refs
        gsem = (gsem0, gsem1)
        asem = (asem0, asem1)
        ci = lax.axis_index("c")
        si = lax.axis_index("s")
        r0 = si * RPT
        if edge_split:
            n_steps = EROWS // (NC * NS)
            base = (ci * NS + si) * n_steps
        else:
            n_steps = OUTER
            base = si * OUTER
        pltpu.sync_copy(za_hbm.at[pl.ds(r0, RPT)], acc_sh.at[pl.ds(r0, RPT)])
        if with_cnt:
            pltpu.sync_copy(zc_hbm.at[pl.ds(r0, RPT)], cnt_sh.at[pl.ds(r0, RPT)])
            pltpu.sync_copy(ones_hbm, ones_v)
        plsc.subcore_barrier()

        def drain(sem, dst_ref, src_ref):
            pltpu.make_async_copy(src_ref, dst_ref, sem).wait()

        def run(tab_hbm):
            def fire_gathers(g, s):
                for j in range(KB):
                    pltpu.async_copy(tab_hbm.at[sidx.at[s, j]],
                                     rows.at[s, j], gsem[s])

            def fire_adds(s):
                for j in range(KB):
                    pltpu.async_copy(rows.at[s, j], acc_sh.at[didx.at[s, j]],
                                     asem[s], add=True)
                if with_cnt:
                    for j in range(KB):
                        pltpu.async_copy(ones_v.at[j],
                                         cnt_sh.at[didx.at[s, j]], csem,
                                         add=True)

            def sub(g, cur, drain_guard, next_guard):
                nxt = 1 - cur

                def start_next():
                    pltpu.sync_copy(src_hbm.at[base + g + 1], sidx.at[nxt])
                    pltpu.sync_copy(dst_hbm.at[base + g + 1], didx.at[nxt])
                    fire_gathers(g + 1, nxt)

                def drain_prev():
                    drain(asem[nxt], rows.at[nxt], dz_hbm)

                if drain_guard is None:
                    drain_prev()
                else:
                    pl.when(drain_guard)(drain_prev)
                if next_guard is None:
                    start_next()
                else:
                    pl.when(next_guard)(start_next)
                drain(gsem[cur], rows.at[cur], dz_hbm)
                fire_adds(cur)

            pltpu.sync_copy(src_hbm.at[base], sidx.at[0])
            pltpu.sync_copy(dst_hbm.at[base], didx.at[0])
            fire_gathers(0, 0)

            def fbody(b, carry):
                sub(2 * b, 0, drain_guard=b > 0, next_guard=None)
                sub(2 * b + 1, 1, drain_guard=None,
                    next_guard=b < n_steps // 2 - 1)
                return carry

            lax.fori_loop(0, n_steps // 2, fbody, 0)
            drain(asem[1], rows.at[1], dz_hbm)

        if edge_split:
            run(t0_hbm)
        else:
            @pl.when(ci == 0)
            def _():
                run(t0_hbm)

            @pl.when(ci == 1)
            def _():
                run(t1_hbm)

        if with_cnt:
            def cdrain(o, carry):
                drain(csem, ones_v, ones_hbm)
                return carry
            lax.fori_loop(0, n_steps, cdrain, 0)

        plsc.subcore_barrier()
        pltpu.sync_copy(acc_sh.at[pl.ds(r0, RPT)],
                        acc_out.at[ci, pl.ds(r0, RPT)])
        if with_cnt:
            pltpu.sync_copy(cnt_sh.at[pl.ds(r0, RPT)],
                            cnt_out.at[ci, pl.ds(r0, RPT)])

    return functools.partial(
        pl.kernel, out_type=out_type, mesh=_MESH, scratch_types=scratch,
        compiler_params=pltpu.CompilerParams(use_tc_tiling_on_sc=False))(body)


_sc_agg1 = _make_sc_agg(F_IN // 2, True)
_sc_agg2 = _make_sc_agg(NCLS, False, edge_split=True)


def _dense1_body(a0_ref, a1_ref, c0_ref, c1_ref, x_ref, w1l_ref, w1r_ref,
                 b1_ref, w2_ref, b2_ref, p_ref, q_ref, inv_ref):
    cnt = (c0_ref[0, :, 0:1] + c1_ref[0, :, 0:1]) * 0.5
    inv = 1.0 / jnp.maximum(cnt, 1.0)
    agg = jnp.concatenate([a0_ref[0], a1_ref[0]], axis=1)
    mean = agg * inv
    h = jnp.maximum(
        jnp.dot(mean, w1l_ref[...], preferred_element_type=jnp.float32)
        + jnp.dot(x_ref[...], w1r_ref[...], preferred_element_type=jnp.float32)
        + b1_ref[...], 0.0)
    pq = jnp.dot(h, w2_ref[...], preferred_element_type=jnp.float32)
    p_ref[...] = pq[:, :NCLS]
    q_ref[...] = pq[:, NCLS:] + b2_ref[...]
    inv_ref[...] = jnp.broadcast_to(inv, (BLK, 8))


def _dense1(acc_parts, cnt_parts, x, W1_l, W1_r, b1, W2cat, b2):
    grid = (N // BLK,)
    return pl.pallas_call(
        _dense1_body,
        grid=grid,
        in_specs=[
            pl.BlockSpec((1, BLK, F_IN // 2), lambda i: (0, i, 0)),
            pl.BlockSpec((1, BLK, F_IN // 2), lambda i: (1, i, 0)),
            pl.BlockSpec((1, BLK, 8), lambda i: (0, i, 0)),
            pl.BlockSpec((1, BLK, 8), lambda i: (1, i, 0)),
            pl.BlockSpec((BLK, F_IN), lambda i: (i, 0)),
            pl.BlockSpec((F_IN, HID), lambda i: (0, 0)),
            pl.BlockSpec((F_IN, HID), lambda i: (0, 0)),
            pl.BlockSpec((1, HID), lambda i: (0, 0)),
            pl.BlockSpec((HID, 2 * NCLS), lambda i: (0, 0)),
            pl.BlockSpec((1, NCLS), lambda i: (0, 0)),
        ],
        out_specs=[
            pl.BlockSpec((BLK, NCLS), lambda i: (i, 0)),
            pl.BlockSpec((BLK, NCLS), lambda i: (i, 0)),
            pl.BlockSpec((BLK, 8), lambda i: (i, 0)),
        ],
        out_shape=[
            jax.ShapeDtypeStruct((N, NCLS), jnp.float32),
            jax.ShapeDtypeStruct((N, NCLS), jnp.float32),
            jax.ShapeDtypeStruct((N, 8), jnp.float32),
        ],
    )(acc_parts, acc_parts, cnt_parts, cnt_parts, x, W1_l, W1_r, b1, W2cat, b2)


def _dense2_body(a0_ref, a1_ref, inv_ref, q_ref, out_ref):
    agg = a0_ref[0] + a1_ref[0]
    z = agg * inv_ref[:, 0:1] + q_ref[...]
    m = jnp.max(z, axis=1, keepdims=True)
    s = jnp.sum(jnp.exp(z - m), axis=1, keepdims=True)
    out_ref[...] = z - m - jnp.log(s)


def _dense2(agg2_parts, inv, q):
    grid = (N // BLK,)
    return pl.pallas_call(
        _dense2_body,
        grid=grid,
        in_specs=[
            pl.BlockSpec((1, BLK, NCLS), lambda i: (0, i, 0)),
            pl.BlockSpec((1, BLK, NCLS), lambda i: (1, i, 0)),
            pl.BlockSpec((BLK, 8), lambda i: (i, 0)),
            pl.BlockSpec((BLK, NCLS), lambda i: (i, 0)),
        ],
        out_specs=pl.BlockSpec((BLK, NCLS), lambda i: (i, 0)),
        out_shape=jax.ShapeDtypeStruct((N, NCLS), jnp.float32),
    )(agg2_parts, agg2_parts, inv, q)


def kernel(x, edge_index, W1_l, W1_r, b1, W2_l, W2_r, b2):
    src = edge_index[0].reshape(EROWS, KB, C)
    dst = edge_index[1].reshape(EROWS, KB, C)

    x0 = x[:, :F_IN // 2]
    x1 = x[:, F_IN // 2:]

    za = jnp.zeros((NP, F_IN // 2), jnp.float32)
    dza = jnp.zeros((KB, C, F_IN // 2), jnp.float32)
    zc = jnp.zeros((NP, 8), jnp.float32)
    zp = jnp.zeros((NP, NCLS), jnp.float32)
    dzp = jnp.zeros((KB, C, NCLS), jnp.float32)
    ones = jnp.ones((KB, C, 8), jnp.float32)

    acc_parts, cnt_parts = jax.tree.leaves(
        _sc_agg1(x0, x1, src, dst, za, dza, zc, ones))

    W2cat = jnp.concatenate([W2_l, W2_r], axis=1)
    p, q, inv = _dense1(acc_parts, cnt_parts, x, W1_l, W1_r, b1[None, :],
                        W2cat, b2[None, :])

    agg2_parts, = jax.tree.leaves(_sc_agg2(p, src, dst, zp, dzp))

    return _dense2(agg2_parts, inv, q)
